# baseline (speedup 1.0000x reference)
import jax
import jax.numpy as jnp
from jax.experimental import pallas as pl
from jax.experimental.pallas import tpu as pltpu

_B, _N = 4, 4096
_TQ = 1024


def _nn_icp_body(last_ref, now_ref, quat_ref, transf_ref, beta_ref,
                 out_ref, kd_ref, kr_ref):
    b = pl.program_id(0)
    q = pl.program_id(1)

    @pl.when((b == 0) & (q == 0))
    def _init():
        quat = quat_ref[...]
        tr = transf_ref[...]
        beta = beta_ref[0, 0]
        dx = tr - jnp.clip(tr, -10.0, 10.0)
        loss_x = jnp.sum(dx * dx) * (1.0 / (_B * 3))
        dq1 = quat[:, :2, :] - jnp.clip(quat[:, :2, :], -15.0, 15.0)
        loss_q1 = jnp.sum(dq1 * dq1) * (1.0 / (_B * 2 * 3))
        dq2 = quat[:, 2, :] - jnp.clip(quat[:, 2, :], -15.0, 15.0)
        loss_q2 = jnp.sum(dq2 * dq2) * (1.0 / (_B * 3))
        out_ref[...] = (loss_x + (loss_q1 + loss_q2) * beta).reshape(1, 1)

    @pl.when(q == 0)
    def _build_keys():
        lastT = jnp.transpose(last_ref[0], (1, 0))
        vl = lastT[0:3, :]
        nl = lastT[3:6, :]
        kd_ref[0:3, :] = -2.0 * vl
        kd_ref[3:4, :] = jnp.sum(vl * vl, axis=0, keepdims=True)
        kr_ref[0:3, :] = nl
        kr_ref[3:4, :] = -jnp.sum(vl * nl, axis=0, keepdims=True)

    vm = now_ref[0, pl.ds(q * _TQ, _TQ), 0:3]
    rota_b = quat_ref[pl.ds(b, 1)][0]
    rotaT = jnp.transpose(rota_b, (1, 0))
    tr_b = transf_ref[pl.ds(b, 1)]
    p = jnp.dot(vm, rotaT, preferred_element_type=jnp.float32)
    p = p + tr_b
    paug = jnp.concatenate([p, jnp.ones((_TQ, 1), jnp.float32)], axis=1)

    dmat = jnp.dot(paug, kd_ref[...], preferred_element_type=jnp.float32)
    rmat = jnp.dot(paug.astype(jnp.bfloat16), kr_ref[...].astype(jnp.bfloat16),
                   preferred_element_type=jnp.float32)
    mrun = jnp.full((_TQ, 128), jnp.inf, jnp.float32)
    rrun = jnp.zeros((_TQ, 128), jnp.float32)
    for c in range(0, _N, 128):
        dc = dmat[:, c:c + 128]
        rc = rmat[:, c:c + 128]
        mask = dc < mrun
        rrun = jnp.where(mask, rc, rrun)
        mrun = jnp.minimum(mrun, dc)
    mf = jnp.min(mrun, axis=1, keepdims=True)
    r = jnp.sum(jnp.where(mrun == mf, rrun, 0.0), axis=1)
    out_ref[...] += jnp.sum(jnp.abs(r)).reshape(1, 1)


def kernel(last_lossalldata, now_lossalldata, quat, trans, sx, sq, beta,
           bindex, needgtloss, rotainput):
    beta2 = beta.reshape(1, 1)
    nq = _N // _TQ
    out = pl.pallas_call(
        _nn_icp_body,
        grid=(_B, nq),
        in_specs=[
            pl.BlockSpec((1, _N, 6), lambda b, q: (b, 0, 0)),
            pl.BlockSpec((1, _N, 6), lambda b, q: (b, 0, 0)),
            pl.BlockSpec((_B, 3, 3), lambda b, q: (0, 0, 0)),
            pl.BlockSpec((_B, 3), lambda b, q: (0, 0)),
            pl.BlockSpec((1, 1), lambda b, q: (0, 0)),
        ],
        out_specs=pl.BlockSpec((1, 1), lambda b, q: (0, 0)),
        out_shape=jax.ShapeDtypeStruct((1, 1), jnp.float32),
        scratch_shapes=[
            pltpu.VMEM((4, _N), jnp.float32),
            pltpu.VMEM((4, _N), jnp.float32),
        ],
        compiler_params=pltpu.CompilerParams(
            dimension_semantics=("arbitrary", "arbitrary")),
    )(last_lossalldata, now_lossalldata, quat, trans, beta2)
    return out[0, 0]
